# Initial kernel scaffold; baseline (speedup 1.0000x reference)
import jax
import jax.numpy as jnp
from jax.experimental import pallas as pl


def kernel(index, value, field, emb_table, field_table, W, b):
    raise NotImplementedError("write your pallas kernel here")


# speedup vs baseline: 2.6851x; 2.6851x over previous
import functools

import jax
import jax.numpy as jnp
from jax import lax
from jax.experimental import pallas as pl
from jax.experimental.pallas import tpu as pltpu
from jax.experimental.pallas import tpu_sc as plsc

B, L = 4096, 100
LP = 112
H = 64
NC, NS, LANES = 2, 16, 16
NW = NC * NS
BPW = B // NW
FT_ROWS = 101
GL = 104


def _sc_kernel(index_hbm, val_hbm, field_hbm, emb_hbm, ft_hbm, wv_hbm,
               out_hbm, idx_v, val_v, field_v, rows_v, ft_v, wv_v,
               fdot_v, out_v, sem):
    wid = lax.axis_index("s") * NC + lax.axis_index("c")
    base = wid * BPW

    pltpu.sync_copy(index_hbm.at[pl.ds(base, BPW)], idx_v)
    pltpu.sync_copy(val_hbm.at[pl.ds(base * LP, BPW * LP)], val_v)
    pltpu.sync_copy(field_hbm.at[pl.ds(base * LP, BPW * LP)], field_v)
    pltpu.sync_copy(ft_hbm, ft_v)
    pltpu.sync_copy(wv_hbm, wv_v)

    lanes = lax.iota(jnp.int32, LANES)

    for c7 in range(LP // LANES):
        row = jnp.minimum(c7 * LANES + lanes, FT_ROWS - 1)

        def fdot_body(j, acc):
            g = plsc.load_gather(ft_v, [row * H + j])
            w2 = plsc.load_gather(wv_v, [jnp.full((LANES,), H + j, jnp.int32)])
            return acc + g * w2

        accf = lax.fori_loop(0, H, fdot_body, jnp.zeros((LANES,), jnp.float32))
        fdot_v[pl.ds(c7 * LANES, LANES)] = accf

    bias_sp = plsc.load_gather(wv_v, [jnp.full((LANES,), 2 * H, jnp.int32)])
    bias_lane0 = jnp.where(lanes == 0, bias_sp, 0.0)
    w1 = [wv_v[pl.ds(c * LANES, LANES)] for c in range(H // LANES)]

    def b_body(b, _):
        pltpu.async_copy(emb_hbm.at[idx_v.at[b, pl.ds(0, GL)]], rows_v,
                         sem).wait()

        def l_body(l, accs):
            vsp = plsc.load_gather(
                val_v, [jnp.full((LANES,), b * LP + l, jnp.int32)])
            return tuple(
                a + vsp * rows_v[l, pl.ds(c * LANES, LANES)]
                for c, a in enumerate(accs))

        zeros = jnp.zeros((LANES,), jnp.float32)
        accs = lax.fori_loop(0, L, l_body, (zeros,) * (H // LANES))

        facc = bias_lane0
        for c in range(LP // LANES):
            off = b * LP + c * LANES
            fvals = val_v[pl.ds(off, LANES)]
            fidx = field_v[pl.ds(off, LANES)]
            facc = facc + fvals * plsc.load_gather(fdot_v, [fidx])

        s = jnp.sum(facc, axis=0)
        for c in range(H // LANES):
            s = s + jnp.sum(accs[c] * w1[c], axis=0)

        plsc.store_scatter(out_v, [jnp.full((LANES,), b, jnp.int32)],
                           jnp.full((LANES,), s, jnp.float32),
                           mask=lanes == 0)
        return 0

    lax.fori_loop(0, BPW, b_body, 0)
    pltpu.sync_copy(out_v, out_hbm.at[pl.ds(base, BPW)])


@functools.partial(jax.jit, static_argnames=())
def _run(index_p, val_flat, field_flat, emb_table, ft_flat, wv):
    mesh = plsc.VectorSubcoreMesh(core_axis_name="c", subcore_axis_name="s")
    k = functools.partial(
        pl.kernel,
        out_type=jax.ShapeDtypeStruct((B,), jnp.float32),
        mesh=mesh,
        compiler_params=pltpu.CompilerParams(needs_layout_passes=False,
                                             use_tc_tiling_on_sc=False),
        scratch_types=[
            pltpu.VMEM((BPW, LP), jnp.int32),
            pltpu.VMEM((BPW * LP,), jnp.float32),
            pltpu.VMEM((BPW * LP,), jnp.int32),
            pltpu.VMEM((GL, H), jnp.float32),
            pltpu.VMEM((FT_ROWS * H,), jnp.float32),
            pltpu.VMEM((144,), jnp.float32),
            pltpu.VMEM((LP,), jnp.float32),
            pltpu.VMEM((BPW,), jnp.float32),
            pltpu.SemaphoreType.DMA,
        ],
    )(_sc_kernel)
    return k(index_p, val_flat, field_flat, emb_table, ft_flat, wv)


def kernel(index, value, field, emb_table, field_table, W, b):
    pad = ((0, 0), (0, LP - L))
    index_p = jnp.pad(index.astype(jnp.int32), pad)
    val_flat = jnp.pad(value, pad).reshape(-1)
    field_flat = jnp.pad(field.astype(jnp.int32), pad).reshape(-1)
    ft_flat = field_table.reshape(-1)
    wv = jnp.concatenate([W[:, 0], b, jnp.zeros((15,), jnp.float32)])
    return _run(index_p, val_flat, field_flat, emb_table, ft_flat, wv)


# speedup vs baseline: 3.7386x; 1.3924x over previous
import functools

import jax
import jax.numpy as jnp
from jax import lax
from jax.experimental import pallas as pl
from jax.experimental.pallas import tpu as pltpu
from jax.experimental.pallas import tpu_sc as plsc

B, L = 4096, 100
LP = 112
H = 64
NC, NS, LANES = 2, 16, 16
NW = NC * NS
BPW = B // NW
FT_ROWS = 101
GL = 104


def _sc_kernel(index_hbm, val_hbm, field_hbm, emb_hbm, ft_hbm, wv_hbm,
               out_hbm, idx_v, val_v, field_v, rows_a, rows_b, ft_v, wv_v,
               fdot_v, out_v, sem, sem2):
    wid = lax.axis_index("s") * NC + lax.axis_index("c")
    base = wid * BPW

    pltpu.sync_copy(index_hbm.at[pl.ds(base, BPW)], idx_v)
    pltpu.sync_copy(val_hbm.at[pl.ds(base * LP, BPW * LP)], val_v)
    pltpu.sync_copy(field_hbm.at[pl.ds(base * LP, BPW * LP)], field_v)
    pltpu.sync_copy(ft_hbm, ft_v)
    pltpu.sync_copy(wv_hbm, wv_v)

    lanes = lax.iota(jnp.int32, LANES)

    for c7 in range(LP // LANES):
        row = jnp.minimum(c7 * LANES + lanes, FT_ROWS - 1)

        def fdot_body(j, acc):
            g = plsc.load_gather(ft_v, [row * H + j])
            w2 = plsc.load_gather(wv_v, [jnp.full((LANES,), H + j, jnp.int32)])
            return acc + g * w2

        accf = lax.fori_loop(0, H, fdot_body, jnp.zeros((LANES,), jnp.float32))
        fdot_v[pl.ds(c7 * LANES, LANES)] = accf

    bias_sp = plsc.load_gather(wv_v, [jnp.full((LANES,), 2 * H, jnp.int32)])
    bias_lane0 = jnp.where(lanes == 0, bias_sp, 0.0)
    w1 = [wv_v[pl.ds(c * LANES, LANES)] for c in range(H // LANES)]

    def compute_row(b, rows):
        def l_body(l, accs):
            vsp = plsc.load_gather(
                val_v, [jnp.full((LANES,), b * LP + l, jnp.int32)])
            return tuple(
                a + vsp * rows[l, pl.ds(c * LANES, LANES)]
                for c, a in enumerate(accs))

        zeros = jnp.zeros((LANES,), jnp.float32)
        accs = lax.fori_loop(0, L, l_body, (zeros,) * (H // LANES),
                             unroll=4)

        facc = bias_lane0
        for c in range(LP // LANES):
            off = b * LP + c * LANES
            fvals = val_v[pl.ds(off, LANES)]
            fidx = field_v[pl.ds(off, LANES)]
            facc = facc + fvals * plsc.load_gather(fdot_v, [fidx])

        s = jnp.sum(facc, axis=0)
        for c in range(H // LANES):
            s = s + jnp.sum(accs[c] * w1[c], axis=0)

        plsc.store_scatter(out_v, [jnp.full((LANES,), b, jnp.int32)],
                           jnp.full((LANES,), s, jnp.float32),
                           mask=lanes == 0)

    def gather_row(b, rows, sem_):
        pltpu.async_copy(emb_hbm.at[idx_v.at[b, pl.ds(0, GL)]], rows, sem_)

    def wait_rows(rows, sem_):
        pltpu.make_async_copy(emb_hbm.at[idx_v.at[0, pl.ds(0, GL)]],
                              rows, sem_).wait()

    gather_row(0, rows_a, sem)
    gather_row(1, rows_b, sem2)

    def b_body(i, _):
        b0 = 2 * i

        wait_rows(rows_a, sem)
        compute_row(b0, rows_a)

        @pl.when(b0 + 2 < BPW)
        def _():
            gather_row(b0 + 2, rows_a, sem)

        wait_rows(rows_b, sem2)
        compute_row(b0 + 1, rows_b)

        @pl.when(b0 + 3 < BPW)
        def _():
            gather_row(b0 + 3, rows_b, sem2)

        return 0

    lax.fori_loop(0, BPW // 2, b_body, 0)
    pltpu.sync_copy(out_v, out_hbm.at[pl.ds(base, BPW)])


@functools.partial(jax.jit, static_argnames=())
def _run(index_p, val_flat, field_flat, emb_table, ft_flat, wv):
    mesh = plsc.VectorSubcoreMesh(core_axis_name="c", subcore_axis_name="s")
    k = functools.partial(
        pl.kernel,
        out_type=jax.ShapeDtypeStruct((B,), jnp.float32),
        mesh=mesh,
        compiler_params=pltpu.CompilerParams(needs_layout_passes=False,
                                             use_tc_tiling_on_sc=False),
        scratch_types=[
            pltpu.VMEM((BPW, LP), jnp.int32),
            pltpu.VMEM((BPW * LP,), jnp.float32),
            pltpu.VMEM((BPW * LP,), jnp.int32),
            pltpu.VMEM((GL, H), jnp.float32),
            pltpu.VMEM((GL, H), jnp.float32),
            pltpu.VMEM((FT_ROWS * H,), jnp.float32),
            pltpu.VMEM((144,), jnp.float32),
            pltpu.VMEM((LP,), jnp.float32),
            pltpu.VMEM((BPW,), jnp.float32),
            pltpu.SemaphoreType.DMA,
            pltpu.SemaphoreType.DMA,
        ],
    )(_sc_kernel)
    return k(index_p, val_flat, field_flat, emb_table, ft_flat, wv)


def kernel(index, value, field, emb_table, field_table, W, b):
    pad = ((0, 0), (0, LP - L))
    spread = (jnp.arange(B * (LP - L), dtype=jnp.int32)
              % emb_table.shape[0]).reshape(B, LP - L)
    index_p = jnp.concatenate([index.astype(jnp.int32), spread], axis=1)
    val_flat = jnp.pad(value, pad).reshape(-1)
    field_flat = jnp.pad(field.astype(jnp.int32), pad).reshape(-1)
    ft_flat = field_table.reshape(-1)
    wv = jnp.concatenate([W[:, 0], b, jnp.zeros((15,), jnp.float32)])
    return _run(index_p, val_flat, field_flat, emb_table, ft_flat, wv)


# speedup vs baseline: 3.9121x; 1.0464x over previous
import functools

import jax
import jax.numpy as jnp
from jax import lax
from jax.experimental import pallas as pl
from jax.experimental.pallas import tpu as pltpu
from jax.experimental.pallas import tpu_sc as plsc

B, L = 4096, 100
LP = 112
H = 64
NC, NS, LANES = 2, 16, 16
NW = NC * NS
BPW = B // NW
FT_ROWS = 101
GL = 104


def _sc_kernel(index_hbm, val_hbm, field_hbm, emb_hbm, ft_hbm, wv_hbm,
               out_hbm, idx_v, val_v, field_v, rows_0, rows_1, rows_2,
               rows_3, ft_v, wv_v, fdot_v, out_v, sem_0, sem_1, sem_2,
               sem_3):
    rows_bufs = (rows_0, rows_1, rows_2, rows_3)
    sems = (sem_0, sem_1, sem_2, sem_3)
    wid = lax.axis_index("s") * NC + lax.axis_index("c")
    base = wid * BPW

    pltpu.sync_copy(index_hbm.at[pl.ds(base, BPW)], idx_v)
    pltpu.sync_copy(val_hbm.at[pl.ds(base * LP, BPW * LP)], val_v)
    pltpu.sync_copy(field_hbm.at[pl.ds(base * LP, BPW * LP)], field_v)
    pltpu.sync_copy(ft_hbm, ft_v)
    pltpu.sync_copy(wv_hbm, wv_v)

    lanes = lax.iota(jnp.int32, LANES)

    for c7 in range(LP // LANES):
        row = jnp.minimum(c7 * LANES + lanes, FT_ROWS - 1)

        def fdot_body(j, acc):
            g = plsc.load_gather(ft_v, [row * H + j])
            w2 = plsc.load_gather(wv_v, [jnp.full((LANES,), H + j, jnp.int32)])
            return acc + g * w2

        accf = lax.fori_loop(0, H, fdot_body, jnp.zeros((LANES,), jnp.float32))
        fdot_v[pl.ds(c7 * LANES, LANES)] = accf

    bias_sp = plsc.load_gather(wv_v, [jnp.full((LANES,), 2 * H, jnp.int32)])
    bias_lane0 = jnp.where(lanes == 0, bias_sp, 0.0)
    w1 = [wv_v[pl.ds(c * LANES, LANES)] for c in range(H // LANES)]

    def compute_row(b, rows):
        def l_body(l, accs):
            vsp = plsc.load_gather(
                val_v, [jnp.full((LANES,), b * LP + l, jnp.int32)])
            return tuple(
                a + vsp * rows[l, pl.ds(c * LANES, LANES)]
                for c, a in enumerate(accs))

        zeros = jnp.zeros((LANES,), jnp.float32)
        accs = lax.fori_loop(0, L, l_body, (zeros,) * (H // LANES),
                             unroll=4)

        facc = bias_lane0
        for c in range(LP // LANES):
            off = b * LP + c * LANES
            fvals = val_v[pl.ds(off, LANES)]
            fidx = field_v[pl.ds(off, LANES)]
            facc = facc + fvals * plsc.load_gather(fdot_v, [fidx])

        s = jnp.sum(facc, axis=0)
        for c in range(H // LANES):
            s = s + jnp.sum(accs[c] * w1[c], axis=0)

        plsc.store_scatter(out_v, [jnp.full((LANES,), b, jnp.int32)],
                           jnp.full((LANES,), s, jnp.float32),
                           mask=lanes == 0)

    def gather_row(b, rows, sem_):
        pltpu.async_copy(emb_hbm.at[idx_v.at[b, pl.ds(0, GL)]], rows, sem_)

    def wait_rows(rows, sem_):
        pltpu.make_async_copy(emb_hbm.at[idx_v.at[0, pl.ds(0, GL)]],
                              rows, sem_).wait()

    NBUF = len(rows_bufs)
    for k in range(NBUF):
        gather_row(k, rows_bufs[k], sems[k])

    def b_body(i, _):
        b0 = NBUF * i
        for k in range(NBUF):
            wait_rows(rows_bufs[k], sems[k])
            compute_row(b0 + k, rows_bufs[k])

            @pl.when(b0 + k + NBUF < BPW)
            def _():
                gather_row(b0 + k + NBUF, rows_bufs[k], sems[k])

        return 0

    lax.fori_loop(0, BPW // NBUF, b_body, 0)
    pltpu.sync_copy(out_v, out_hbm.at[pl.ds(base, BPW)])


@functools.partial(jax.jit, static_argnames=())
def _run(index_p, val_flat, field_flat, emb_table, ft_flat, wv):
    mesh = plsc.VectorSubcoreMesh(core_axis_name="c", subcore_axis_name="s")
    k = functools.partial(
        pl.kernel,
        out_type=jax.ShapeDtypeStruct((B,), jnp.float32),
        mesh=mesh,
        compiler_params=pltpu.CompilerParams(needs_layout_passes=False,
                                             use_tc_tiling_on_sc=False),
        scratch_types=[
            pltpu.VMEM((BPW, LP), jnp.int32),
            pltpu.VMEM((BPW * LP,), jnp.float32),
            pltpu.VMEM((BPW * LP,), jnp.int32),
            pltpu.VMEM((GL, H), jnp.float32),
            pltpu.VMEM((GL, H), jnp.float32),
            pltpu.VMEM((GL, H), jnp.float32),
            pltpu.VMEM((GL, H), jnp.float32),
            pltpu.VMEM((FT_ROWS * H,), jnp.float32),
            pltpu.VMEM((144,), jnp.float32),
            pltpu.VMEM((LP,), jnp.float32),
            pltpu.VMEM((BPW,), jnp.float32),
            pltpu.SemaphoreType.DMA,
            pltpu.SemaphoreType.DMA,
            pltpu.SemaphoreType.DMA,
            pltpu.SemaphoreType.DMA,
        ],
    )(_sc_kernel)
    return k(index_p, val_flat, field_flat, emb_table, ft_flat, wv)


def kernel(index, value, field, emb_table, field_table, W, b):
    pad = ((0, 0), (0, LP - L))
    spread = (jnp.arange(B * (LP - L), dtype=jnp.int32)
              % emb_table.shape[0]).reshape(B, LP - L)
    index_p = jnp.concatenate([index.astype(jnp.int32), spread], axis=1)
    val_flat = jnp.pad(value, pad).reshape(-1)
    field_flat = jnp.pad(field.astype(jnp.int32), pad).reshape(-1)
    ft_flat = field_table.reshape(-1)
    wv = jnp.concatenate([W[:, 0], b, jnp.zeros((15,), jnp.float32)])
    return _run(index_p, val_flat, field_flat, emb_table, ft_flat, wv)


# speedup vs baseline: 4.5009x; 1.1505x over previous
import functools

import jax
import jax.numpy as jnp
from jax import lax
from jax.experimental import pallas as pl
from jax.experimental.pallas import tpu as pltpu
from jax.experimental.pallas import tpu_sc as plsc

B, L = 4096, 100
LP = 112
H = 64
NC, NS, LANES = 2, 16, 16
NW = NC * NS
BPW = B // NW
FT_ROWS = 101
GL = LP
NBUF = 4

BK = 8192
NBLK = 123
EPAD = NBLK * BK


def _edot_body(w_ref, x_ref, o_ref):
    r = lax.dot_general(w_ref[...], x_ref[...], (((0,), (1,)), ((), ())),
                        preferred_element_type=jnp.float32)
    o_ref[...] = r[0:1, :].reshape(1, 1, BK)


def _tc_edot(emb_table, wmat):
    return pl.pallas_call(
        _edot_body,
        grid=(NBLK,),
        in_specs=[
            pl.BlockSpec((H, 128), lambda i: (0, 0)),
            pl.BlockSpec((BK, H), lambda i: (i, 0)),
        ],
        out_specs=pl.BlockSpec((1, 1, BK), lambda i: (i, 0, 0)),
        out_shape=jax.ShapeDtypeStruct((NBLK, 1, BK), jnp.float32),
    )(wmat, emb_table)


def _sc_kernel(index_hbm, val_hbm, field_hbm, e_hbm, ft_hbm, wv_hbm,
               out_hbm, idx_v, val_v, field_v, e_sp, eb_0, eb_1, eb_2,
               eb_3, ft_v, wv_v, fdot_v, out_v, sem_0, sem_1, sem_2,
               sem_3):
    e_bufs = (eb_0, eb_1, eb_2, eb_3)
    sems = (sem_0, sem_1, sem_2, sem_3)
    sid = lax.axis_index("s")
    wid = sid * NC + lax.axis_index("c")
    base = wid * BPW

    @pl.when(sid == 0)
    def _():
        pltpu.sync_copy(e_hbm, e_sp)

    pltpu.sync_copy(index_hbm.at[pl.ds(base, BPW)], idx_v)
    pltpu.sync_copy(val_hbm.at[pl.ds(base * LP, BPW * LP)], val_v)
    pltpu.sync_copy(field_hbm.at[pl.ds(base * LP, BPW * LP)], field_v)
    pltpu.sync_copy(ft_hbm, ft_v)
    pltpu.sync_copy(wv_hbm, wv_v)

    lanes = lax.iota(jnp.int32, LANES)
    zeros = jnp.zeros((LANES,), jnp.float32)

    for c7 in range(LP // LANES):
        row = jnp.minimum(c7 * LANES + lanes, FT_ROWS - 1)

        def fdot_body(j, acc):
            g = plsc.load_gather(ft_v, [row * H + j])
            w2 = plsc.load_gather(wv_v, [jnp.full((LANES,), H + j, jnp.int32)])
            return acc + g * w2

        accf = lax.fori_loop(0, H, fdot_body, zeros)
        fdot_v[pl.ds(c7 * LANES, LANES)] = accf

    bias_sp = plsc.load_gather(wv_v, [jnp.full((LANES,), 2 * H, jnp.int32)])
    bias_lane0 = jnp.where(lanes == 0, bias_sp, 0.0)

    plsc.subcore_barrier()

    def gather_e(b, eb, sem_):
        pltpu.async_copy(e_sp.at[idx_v.at[b]], eb, sem_)

    def wait_e(eb, sem_):
        pltpu.make_async_copy(e_sp.at[idx_v.at[0]], eb, sem_).wait()

    def compute_row(b, eb):
        facc = bias_lane0
        eacc = zeros
        for c in range(LP // LANES):
            off = b * LP + c * LANES
            v = val_v[pl.ds(off, LANES)]
            fidx = field_v[pl.ds(off, LANES)]
            facc = facc + v * plsc.load_gather(fdot_v, [fidx])
            eacc = eacc + v * eb[pl.ds(c * LANES, LANES)]

        s = jnp.sum(facc + eacc, axis=0)
        plsc.store_scatter(out_v, [jnp.full((LANES,), b, jnp.int32)],
                           jnp.full((LANES,), s, jnp.float32),
                           mask=lanes == 0)

    for k in range(NBUF):
        gather_e(k, e_bufs[k], sems[k])

    def b_body(i, _):
        b0 = NBUF * i
        for k in range(NBUF):
            wait_e(e_bufs[k], sems[k])
            compute_row(b0 + k, e_bufs[k])

            @pl.when(b0 + k + NBUF < BPW)
            def _():
                gather_e(b0 + k + NBUF, e_bufs[k], sems[k])

        return 0

    lax.fori_loop(0, BPW // NBUF, b_body, 0)
    pltpu.sync_copy(out_v, out_hbm.at[pl.ds(base, BPW)])


@jax.jit
def _run(index_p, val_flat, field_flat, e_flat, ft_flat, wv):
    mesh = plsc.VectorSubcoreMesh(core_axis_name="c", subcore_axis_name="s")
    k = functools.partial(
        pl.kernel,
        out_type=jax.ShapeDtypeStruct((B,), jnp.float32),
        mesh=mesh,
        compiler_params=pltpu.CompilerParams(needs_layout_passes=False,
                                             use_tc_tiling_on_sc=False),
        scratch_types=[
            pltpu.VMEM((BPW, LP), jnp.int32),
            pltpu.VMEM((BPW * LP,), jnp.float32),
            pltpu.VMEM((BPW * LP,), jnp.int32),
            pltpu.VMEM_SHARED((EPAD,), jnp.float32),
            pltpu.VMEM((LP,), jnp.float32),
            pltpu.VMEM((LP,), jnp.float32),
            pltpu.VMEM((LP,), jnp.float32),
            pltpu.VMEM((LP,), jnp.float32),
            pltpu.VMEM((FT_ROWS * H,), jnp.float32),
            pltpu.VMEM((144,), jnp.float32),
            pltpu.VMEM((LP,), jnp.float32),
            pltpu.VMEM((BPW,), jnp.float32),
            pltpu.SemaphoreType.DMA,
            pltpu.SemaphoreType.DMA,
            pltpu.SemaphoreType.DMA,
            pltpu.SemaphoreType.DMA,
        ],
    )(_sc_kernel)
    return k(index_p, val_flat, field_flat, e_flat, ft_flat, wv)


def kernel(index, value, field, emb_table, field_table, W, b):
    wmat = jnp.zeros((H, 128), jnp.float32).at[:, 0].set(W[:H, 0])
    e_flat = _tc_edot(emb_table, wmat).reshape(-1)

    pad = ((0, 0), (0, LP - L))
    spread = (jnp.arange(B * (LP - L), dtype=jnp.int32)
              % emb_table.shape[0]).reshape(B, LP - L)
    index_p = jnp.concatenate([index.astype(jnp.int32), spread], axis=1)
    val_flat = jnp.pad(value, pad).reshape(-1)
    field_flat = jnp.pad(field.astype(jnp.int32), pad).reshape(-1)
    ft_flat = field_table.reshape(-1)
    wv = jnp.concatenate([W[:, 0], b, jnp.zeros((15,), jnp.float32)])
    return _run(index_p, val_flat, field_flat, e_flat, ft_flat, wv)


# speedup vs baseline: 14.5718x; 3.2376x over previous
import functools

import jax
import jax.numpy as jnp
from jax import lax
from jax.experimental import pallas as pl
from jax.experimental.pallas import tpu as pltpu
from jax.experimental.pallas import tpu_sc as plsc

B, L = 4096, 100
LP = 112
H = 64
NC, NS, LANES = 2, 16, 16
NW = NC * NS
BPW = B // NW
FT_ROWS = 101
GL = LP
NBUF = 4

BK = 8192
NBLK = 123
EPAD = NBLK * BK


def _edot_body(w_ref, x_ref, o_ref):
    r = lax.dot_general(w_ref[...], x_ref[...], (((1,), (0,)), ((), ())),
                        preferred_element_type=jnp.float32)
    o_ref[...] = r[0:1, :].reshape(1, 1, BK)


def _tc_edot(emb_t, wmat):
    return pl.pallas_call(
        _edot_body,
        grid=(NBLK,),
        in_specs=[
            pl.BlockSpec((8, H), lambda i: (0, 0)),
            pl.BlockSpec((H, BK), lambda i: (0, i)),
        ],
        out_specs=pl.BlockSpec((1, 1, BK), lambda i: (i, 0, 0)),
        out_shape=jax.ShapeDtypeStruct((NBLK, 1, BK), jnp.float32),
    )(wmat, emb_t)


def _sc_kernel(index_hbm, val_hbm, field_hbm, e_hbm, ft_hbm, wv_hbm,
               out_hbm, idx_v, val_v, field_v, e_sp, eb_0, eb_1, eb_2,
               eb_3, ft_v, wv_v, fdot_v, out_v, sem_0, sem_1, sem_2,
               sem_3):
    e_bufs = (eb_0, eb_1, eb_2, eb_3)
    sems = (sem_0, sem_1, sem_2, sem_3)
    sid = lax.axis_index("s")
    wid = sid * NC + lax.axis_index("c")
    base = wid * BPW

    @pl.when(sid == 0)
    def _():
        pltpu.sync_copy(e_hbm, e_sp)

    pltpu.sync_copy(index_hbm.at[pl.ds(base, BPW)], idx_v)
    pltpu.sync_copy(val_hbm.at[pl.ds(base * LP, BPW * LP)], val_v)
    pltpu.sync_copy(field_hbm.at[pl.ds(base * LP, BPW * LP)], field_v)
    pltpu.sync_copy(ft_hbm, ft_v)
    pltpu.sync_copy(wv_hbm, wv_v)

    lanes = lax.iota(jnp.int32, LANES)
    zeros = jnp.zeros((LANES,), jnp.float32)

    for c7 in range(LP // LANES):
        row = jnp.minimum(c7 * LANES + lanes, FT_ROWS - 1)

        def fdot_body(j, acc):
            g = plsc.load_gather(ft_v, [row * H + j])
            w2 = plsc.load_gather(wv_v, [jnp.full((LANES,), H + j, jnp.int32)])
            return acc + g * w2

        accf = lax.fori_loop(0, H, fdot_body, zeros)
        fdot_v[pl.ds(c7 * LANES, LANES)] = accf

    bias_sp = plsc.load_gather(wv_v, [jnp.full((LANES,), 2 * H, jnp.int32)])
    bias_lane0 = jnp.where(lanes == 0, bias_sp, 0.0)

    plsc.subcore_barrier()

    def gather_e(b, eb, sem_):
        pltpu.async_copy(e_sp.at[idx_v.at[b]], eb, sem_)

    def wait_e(eb, sem_):
        pltpu.make_async_copy(e_sp.at[idx_v.at[0]], eb, sem_).wait()

    def compute_row(b, eb):
        facc = bias_lane0
        eacc = zeros
        for c in range(LP // LANES):
            off = b * LP + c * LANES
            v = val_v[pl.ds(off, LANES)]
            fidx = field_v[pl.ds(off, LANES)]
            facc = facc + v * plsc.load_gather(fdot_v, [fidx])
            eacc = eacc + v * eb[pl.ds(c * LANES, LANES)]

        s = jnp.sum(facc + eacc, axis=0)
        plsc.store_scatter(out_v, [jnp.full((LANES,), b, jnp.int32)],
                           jnp.full((LANES,), s, jnp.float32),
                           mask=lanes == 0)

    for k in range(NBUF):
        gather_e(k, e_bufs[k], sems[k])

    def b_body(i, _):
        b0 = NBUF * i
        for k in range(NBUF):
            wait_e(e_bufs[k], sems[k])
            compute_row(b0 + k, e_bufs[k])

            @pl.when(b0 + k + NBUF < BPW)
            def _():
                gather_e(b0 + k + NBUF, e_bufs[k], sems[k])

        return 0

    lax.fori_loop(0, BPW // NBUF, b_body, 0)
    pltpu.sync_copy(out_v, out_hbm.at[pl.ds(base, BPW)])


@jax.jit
def _run(index_p, val_flat, field_flat, e_flat, ft_flat, wv):
    mesh = plsc.VectorSubcoreMesh(core_axis_name="c", subcore_axis_name="s")
    k = functools.partial(
        pl.kernel,
        out_type=jax.ShapeDtypeStruct((B,), jnp.float32),
        mesh=mesh,
        compiler_params=pltpu.CompilerParams(needs_layout_passes=False,
                                             use_tc_tiling_on_sc=False),
        scratch_types=[
            pltpu.VMEM((BPW, LP), jnp.int32),
            pltpu.VMEM((BPW * LP,), jnp.float32),
            pltpu.VMEM((BPW * LP,), jnp.int32),
            pltpu.VMEM_SHARED((EPAD,), jnp.float32),
            pltpu.VMEM((LP,), jnp.float32),
            pltpu.VMEM((LP,), jnp.float32),
            pltpu.VMEM((LP,), jnp.float32),
            pltpu.VMEM((LP,), jnp.float32),
            pltpu.VMEM((FT_ROWS * H,), jnp.float32),
            pltpu.VMEM((144,), jnp.float32),
            pltpu.VMEM((LP,), jnp.float32),
            pltpu.VMEM((BPW,), jnp.float32),
            pltpu.SemaphoreType.DMA,
            pltpu.SemaphoreType.DMA,
            pltpu.SemaphoreType.DMA,
            pltpu.SemaphoreType.DMA,
        ],
    )(_sc_kernel)
    return k(index_p, val_flat, field_flat, e_flat, ft_flat, wv)


def kernel(index, value, field, emb_table, field_table, W, b):
    wmat = jnp.zeros((8, H), jnp.float32).at[0].set(W[:H, 0])
    e_flat = _tc_edot(emb_table.T, wmat).reshape(-1)

    pad = ((0, 0), (0, LP - L))
    spread = (jnp.arange(B * (LP - L), dtype=jnp.int32)
              % emb_table.shape[0]).reshape(B, LP - L)
    index_p = jnp.concatenate([index.astype(jnp.int32), spread], axis=1)
    val_flat = jnp.pad(value, pad).reshape(-1)
    field_flat = jnp.pad(field.astype(jnp.int32), pad).reshape(-1)
    ft_flat = field_table.reshape(-1)
    wv = jnp.concatenate([W[:, 0], b, jnp.zeros((15,), jnp.float32)])
    return _run(index_p, val_flat, field_flat, e_flat, ft_flat, wv)


# speedup vs baseline: 18.0766x; 1.2405x over previous
import functools

import jax
import jax.numpy as jnp
from jax import lax
from jax.experimental import pallas as pl
from jax.experimental.pallas import tpu as pltpu
from jax.experimental.pallas import tpu_sc as plsc

B, L = 4096, 100
LP = 112
H = 64
NC, NS, LANES = 2, 16, 16
NW = NC * NS
BPW = B // NW
FT_ROWS = 101
GL = LP
NBUF = 4

BK = 16384
NBLK = 62
EPAD = NBLK * BK


def _edot_body(w_ref, x_ref, o_ref):
    r = lax.dot_general(w_ref[...], x_ref[...], (((1,), (0,)), ((), ())),
                        preferred_element_type=jnp.float32)
    o_ref[...] = r[0:1, :].reshape(1, 1, BK)


def _tc_edot(emb_t, wmat):
    return pl.pallas_call(
        _edot_body,
        grid=(NBLK,),
        in_specs=[
            pl.BlockSpec((8, H), lambda i: (0, 0)),
            pl.BlockSpec((H, BK), lambda i: (0, i)),
        ],
        out_specs=pl.BlockSpec((1, 1, BK), lambda i: (i, 0, 0)),
        out_shape=jax.ShapeDtypeStruct((NBLK, 1, BK), jnp.float32),
    )(wmat, emb_t)


def _sc_kernel(index_hbm, val_hbm, field_hbm, e_hbm, ft_hbm, wv_hbm,
               out_hbm, idx_v, val_v, field_v, e_sp, eb_0, eb_1, eb_2,
               eb_3, ft_v, wv_v, fdot_v, out_v, sem_0, sem_1, sem_2,
               sem_3):
    e_bufs = (eb_0, eb_1, eb_2, eb_3)
    sems = (sem_0, sem_1, sem_2, sem_3)
    sid = lax.axis_index("s")
    wid = sid * NC + lax.axis_index("c")
    base = wid * BPW

    @pl.when(sid == 0)
    def _():
        pltpu.sync_copy(e_hbm, e_sp)

    pltpu.sync_copy(index_hbm.at[pl.ds(base, BPW)], idx_v)
    pltpu.sync_copy(val_hbm.at[pl.ds(base * LP, BPW * LP)], val_v)
    pltpu.sync_copy(field_hbm.at[pl.ds(base * LP, BPW * LP)], field_v)
    pltpu.sync_copy(ft_hbm, ft_v)
    pltpu.sync_copy(wv_hbm, wv_v)

    lanes = lax.iota(jnp.int32, LANES)
    zeros = jnp.zeros((LANES,), jnp.float32)

    for c7 in range(LP // LANES):
        row = jnp.minimum(c7 * LANES + lanes, FT_ROWS - 1)

        def fdot_body(j, acc):
            g = plsc.load_gather(ft_v, [row * H + j])
            w2 = plsc.load_gather(wv_v, [jnp.full((LANES,), H + j, jnp.int32)])
            return acc + g * w2

        accf = lax.fori_loop(0, H, fdot_body, zeros)
        fdot_v[pl.ds(c7 * LANES, LANES)] = accf

    bias_sp = plsc.load_gather(wv_v, [jnp.full((LANES,), 2 * H, jnp.int32)])
    bias_lane0 = jnp.where(lanes == 0, bias_sp, 0.0)

    plsc.subcore_barrier()

    def gather_e(b, eb, sem_):
        pltpu.async_copy(e_sp.at[idx_v.at[b]], eb, sem_)

    def wait_e(eb, sem_):
        pltpu.make_async_copy(e_sp.at[idx_v.at[0]], eb, sem_).wait()

    def compute_row(b, eb):
        facc = bias_lane0
        eacc = zeros
        for c in range(LP // LANES):
            off = b * LP + c * LANES
            v = val_v[pl.ds(off, LANES)]
            fidx = field_v[pl.ds(off, LANES)]
            facc = facc + v * plsc.load_gather(fdot_v, [fidx])
            eacc = eacc + v * eb[pl.ds(c * LANES, LANES)]

        s = jnp.sum(facc + eacc, axis=0)
        plsc.store_scatter(out_v, [jnp.full((LANES,), b, jnp.int32)],
                           jnp.full((LANES,), s, jnp.float32),
                           mask=lanes == 0)

    for k in range(NBUF):
        gather_e(k, e_bufs[k], sems[k])

    def b_body(i, _):
        b0 = NBUF * i
        for k in range(NBUF):
            wait_e(e_bufs[k], sems[k])
            compute_row(b0 + k, e_bufs[k])

            @pl.when(b0 + k + NBUF < BPW)
            def _():
                gather_e(b0 + k + NBUF, e_bufs[k], sems[k])

        return 0

    lax.fori_loop(0, BPW // NBUF, b_body, 0)
    pltpu.sync_copy(out_v, out_hbm.at[pl.ds(base, BPW)])


@jax.jit
def _run(index_p, val_flat, field_flat, e_flat, ft_flat, wv):
    mesh = plsc.VectorSubcoreMesh(core_axis_name="c", subcore_axis_name="s")
    k = functools.partial(
        pl.kernel,
        out_type=jax.ShapeDtypeStruct((B,), jnp.float32),
        mesh=mesh,
        compiler_params=pltpu.CompilerParams(needs_layout_passes=False,
                                             use_tc_tiling_on_sc=False),
        scratch_types=[
            pltpu.VMEM((BPW, LP), jnp.int32),
            pltpu.VMEM((BPW * LP,), jnp.float32),
            pltpu.VMEM((BPW * LP,), jnp.int32),
            pltpu.VMEM_SHARED((EPAD,), jnp.float32),
            pltpu.VMEM((LP,), jnp.float32),
            pltpu.VMEM((LP,), jnp.float32),
            pltpu.VMEM((LP,), jnp.float32),
            pltpu.VMEM((LP,), jnp.float32),
            pltpu.VMEM((FT_ROWS * H,), jnp.float32),
            pltpu.VMEM((144,), jnp.float32),
            pltpu.VMEM((LP,), jnp.float32),
            pltpu.VMEM((BPW,), jnp.float32),
            pltpu.SemaphoreType.DMA,
            pltpu.SemaphoreType.DMA,
            pltpu.SemaphoreType.DMA,
            pltpu.SemaphoreType.DMA,
        ],
    )(_sc_kernel)
    return k(index_p, val_flat, field_flat, e_flat, ft_flat, wv)


def kernel(index, value, field, emb_table, field_table, W, b):
    wmat = jnp.zeros((8, H), jnp.float32).at[0].set(W[:H, 0])
    e_flat = _tc_edot(emb_table.T, wmat).reshape(-1)

    pad = ((0, 0), (0, LP - L))
    spread = (jnp.arange(B * (LP - L), dtype=jnp.int32)
              % emb_table.shape[0]).reshape(B, LP - L)
    index_p = jnp.concatenate([index.astype(jnp.int32), spread], axis=1)
    val_flat = jnp.pad(value, pad).reshape(-1)
    field_flat = jnp.pad(field.astype(jnp.int32), pad).reshape(-1)
    ft_flat = field_table.reshape(-1)
    wv = jnp.concatenate([W[:, 0], b, jnp.zeros((15,), jnp.float32)])
    return _run(index_p, val_flat, field_flat, e_flat, ft_flat, wv)


# speedup vs baseline: 19.0194x; 1.0522x over previous
import functools

import jax
import jax.numpy as jnp
from jax import lax
from jax.experimental import pallas as pl
from jax.experimental.pallas import tpu as pltpu
from jax.experimental.pallas import tpu_sc as plsc

B, L = 4096, 100
LP = 112
H = 64
NC, NS, LANES = 2, 16, 16
NW = NC * NS
BPW = B // NW
FT_ROWS = 101
GL = LP
NBUF = 4

BK = 65536
NBLK = 16
EPAD = NBLK * BK


def _edot_body(w_ref, x_ref, o_ref):
    r = lax.dot_general(w_ref[...], x_ref[...], (((1,), (0,)), ((), ())),
                        preferred_element_type=jnp.float32)
    o_ref[...] = r[0:1, :].reshape(1, 1, BK)


def _tc_edot(emb_t, wmat):
    return pl.pallas_call(
        _edot_body,
        grid=(NBLK,),
        in_specs=[
            pl.BlockSpec((8, H), lambda i: (0, 0)),
            pl.BlockSpec((H, BK), lambda i: (0, i)),
        ],
        out_specs=pl.BlockSpec((1, 1, BK), lambda i: (i, 0, 0)),
        out_shape=jax.ShapeDtypeStruct((NBLK, 1, BK), jnp.float32),
    )(wmat, emb_t)


def _sc_kernel(index_hbm, val_hbm, field_hbm, e_hbm, ft_hbm, wv_hbm,
               out_hbm, idx_v, val_v, field_v, e_sp, eb_0, eb_1, eb_2,
               eb_3, ft_v, wv_v, fdot_v, out_v, sem_0, sem_1, sem_2,
               sem_3):
    e_bufs = (eb_0, eb_1, eb_2, eb_3)
    sems = (sem_0, sem_1, sem_2, sem_3)
    sid = lax.axis_index("s")
    wid = sid * NC + lax.axis_index("c")
    base = wid * BPW

    @pl.when(sid == 0)
    def _():
        pltpu.sync_copy(e_hbm, e_sp)

    pltpu.sync_copy(index_hbm.at[pl.ds(base, BPW)], idx_v)
    pltpu.sync_copy(val_hbm.at[pl.ds(base * LP, BPW * LP)], val_v)
    pltpu.sync_copy(field_hbm.at[pl.ds(base * LP, BPW * LP)], field_v)
    pltpu.sync_copy(ft_hbm, ft_v)
    pltpu.sync_copy(wv_hbm, wv_v)

    lanes = lax.iota(jnp.int32, LANES)
    zeros = jnp.zeros((LANES,), jnp.float32)

    for c7 in range(LP // LANES):
        row = jnp.minimum(c7 * LANES + lanes, FT_ROWS - 1)

        def fdot_body(j, acc):
            g = plsc.load_gather(ft_v, [row * H + j])
            w2 = plsc.load_gather(wv_v, [jnp.full((LANES,), H + j, jnp.int32)])
            return acc + g * w2

        accf = lax.fori_loop(0, H, fdot_body, zeros)
        fdot_v[pl.ds(c7 * LANES, LANES)] = accf

    bias_sp = plsc.load_gather(wv_v, [jnp.full((LANES,), 2 * H, jnp.int32)])
    bias_lane0 = jnp.where(lanes == 0, bias_sp, 0.0)

    plsc.subcore_barrier()

    def gather_e(b, eb, sem_):
        pltpu.async_copy(e_sp.at[idx_v.at[b]], eb, sem_)

    def wait_e(eb, sem_):
        pltpu.make_async_copy(e_sp.at[idx_v.at[0]], eb, sem_).wait()

    def compute_row(b, eb):
        facc = bias_lane0
        eacc = zeros
        for c in range(LP // LANES):
            off = b * LP + c * LANES
            v = val_v[pl.ds(off, LANES)]
            fidx = field_v[pl.ds(off, LANES)]
            facc = facc + v * plsc.load_gather(fdot_v, [fidx])
            eacc = eacc + v * eb[pl.ds(c * LANES, LANES)]

        s = jnp.sum(facc + eacc, axis=0)
        plsc.store_scatter(out_v, [jnp.full((LANES,), b, jnp.int32)],
                           jnp.full((LANES,), s, jnp.float32),
                           mask=lanes == 0)

    for k in range(NBUF):
        gather_e(k, e_bufs[k], sems[k])

    def b_body(i, _):
        b0 = NBUF * i
        for k in range(NBUF):
            wait_e(e_bufs[k], sems[k])
            compute_row(b0 + k, e_bufs[k])

            @pl.when(b0 + k + NBUF < BPW)
            def _():
                gather_e(b0 + k + NBUF, e_bufs[k], sems[k])

        return 0

    lax.fori_loop(0, BPW // NBUF, b_body, 0)
    pltpu.sync_copy(out_v, out_hbm.at[pl.ds(base, BPW)])


@jax.jit
def _run(index_p, val_flat, field_flat, e_flat, ft_flat, wv):
    mesh = plsc.VectorSubcoreMesh(core_axis_name="c", subcore_axis_name="s")
    k = functools.partial(
        pl.kernel,
        out_type=jax.ShapeDtypeStruct((B,), jnp.float32),
        mesh=mesh,
        compiler_params=pltpu.CompilerParams(needs_layout_passes=False,
                                             use_tc_tiling_on_sc=False),
        scratch_types=[
            pltpu.VMEM((BPW, LP), jnp.int32),
            pltpu.VMEM((BPW * LP,), jnp.float32),
            pltpu.VMEM((BPW * LP,), jnp.int32),
            pltpu.VMEM_SHARED((EPAD,), jnp.float32),
            pltpu.VMEM((LP,), jnp.float32),
            pltpu.VMEM((LP,), jnp.float32),
            pltpu.VMEM((LP,), jnp.float32),
            pltpu.VMEM((LP,), jnp.float32),
            pltpu.VMEM((FT_ROWS * H,), jnp.float32),
            pltpu.VMEM((144,), jnp.float32),
            pltpu.VMEM((LP,), jnp.float32),
            pltpu.VMEM((BPW,), jnp.float32),
            pltpu.SemaphoreType.DMA,
            pltpu.SemaphoreType.DMA,
            pltpu.SemaphoreType.DMA,
            pltpu.SemaphoreType.DMA,
        ],
    )(_sc_kernel)
    return k(index_p, val_flat, field_flat, e_flat, ft_flat, wv)


def kernel(index, value, field, emb_table, field_table, W, b):
    wmat = jnp.zeros((8, H), jnp.float32).at[0].set(W[:H, 0])
    e_flat = _tc_edot(emb_table.T, wmat).reshape(-1)

    pad = ((0, 0), (0, LP - L))
    spread = (jnp.arange(B * (LP - L), dtype=jnp.int32)
              % emb_table.shape[0]).reshape(B, LP - L)
    index_p = jnp.concatenate([index.astype(jnp.int32), spread], axis=1)
    val_flat = jnp.pad(value, pad).reshape(-1)
    field_flat = jnp.pad(field.astype(jnp.int32), pad).reshape(-1)
    ft_flat = field_table.reshape(-1)
    wv = jnp.concatenate([W[:, 0], b, jnp.zeros((15,), jnp.float32)])
    return _run(index_p, val_flat, field_flat, e_flat, ft_flat, wv)
